# baseline (speedup 1.0000x reference)
import functools
import jax
import jax.numpy as jnp
from jax import lax
from jax.experimental import pallas as pl
from jax.experimental.pallas import tpu as pltpu
from jax.experimental.pallas import tpu_sc as plsc

_NW = 32
_CH = 16
_NSLOTS = 7
_PF = 4


def _sc_body(batch, n_chunks, rows_per_worker, table_hbm, out_hbm, buf,
             in_sem, out_sem):
    wid = lax.axis_index("s") * 2 + lax.axis_index("c")
    base = wid * rows_per_worker

    def in_copy(c):
        return pltpu.make_async_copy(
            table_hbm.at[pl.ds(base + c * _CH, _CH), :],
            buf.at[c % _NSLOTS], in_sem,
        )

    def out_copy(c, b):
        return pltpu.make_async_copy(
            buf.at[c % _NSLOTS],
            out_hbm.at[b, pl.ds(base + c * _CH, _CH), :], out_sem,
        )

    for c in range(min(_PF, n_chunks)):
        in_copy(c).start()
    for c in range(n_chunks):
        in_copy(c).wait()
        for b in range(batch):
            out_copy(c, b).start()
        nxt = c + _PF
        if nxt < n_chunks:
            old = nxt - _NSLOTS
            if old >= 0:
                for b in range(batch):
                    out_copy(old, b).wait()
            in_copy(nxt).start()
    for c in range(max(0, n_chunks - _NSLOTS), n_chunks):
        for b in range(batch):
            out_copy(c, b).wait()


def kernel(x, pos_embedding):
    B, S = x.shape
    H = pos_embedding.shape[1]
    rows_per_worker = S // _NW
    n_chunks = rows_per_worker // _CH
    mesh = plsc.VectorSubcoreMesh(core_axis_name="c", subcore_axis_name="s")
    body = functools.partial(_sc_body, B, n_chunks, rows_per_worker)
    k = functools.partial(
        pl.kernel,
        mesh=mesh,
        out_type=jax.ShapeDtypeStruct((B, S, H), pos_embedding.dtype),
        scratch_types=[
            pltpu.VMEM((_NSLOTS, _CH, H), pos_embedding.dtype),
            pltpu.SemaphoreType.DMA,
            pltpu.SemaphoreType.DMA,
        ],
    )(body)
    return k(pos_embedding)


# speedup vs baseline: 1.0419x; 1.0419x over previous
import functools
import jax
import jax.numpy as jnp
from jax import lax
from jax.experimental import pallas as pl
from jax.experimental.pallas import tpu as pltpu
from jax.experimental.pallas import tpu_sc as plsc

_NW = 32
_CH = 32
_NSLOTS = 3
_PF = 2


def _sc_body(batch, n_chunks, rows_per_worker, table_hbm, out_hbm, buf,
             in_sem, out_sem):
    wid = lax.axis_index("s") * 2 + lax.axis_index("c")
    base = wid * rows_per_worker

    def in_copy(c):
        return pltpu.make_async_copy(
            table_hbm.at[pl.ds(base + c * _CH, _CH), :],
            buf.at[c % _NSLOTS], in_sem,
        )

    def out_copy(c, b):
        return pltpu.make_async_copy(
            buf.at[c % _NSLOTS],
            out_hbm.at[b, pl.ds(base + c * _CH, _CH), :], out_sem,
        )

    for c in range(min(_PF, n_chunks)):
        in_copy(c).start()
    for c in range(n_chunks):
        in_copy(c).wait()
        for b in range(batch):
            out_copy(c, b).start()
        nxt = c + _PF
        if nxt < n_chunks:
            old = nxt - _NSLOTS
            if old >= 0:
                for b in range(batch):
                    out_copy(old, b).wait()
            in_copy(nxt).start()
    for c in range(max(0, n_chunks - _NSLOTS), n_chunks):
        for b in range(batch):
            out_copy(c, b).wait()


def kernel(x, pos_embedding):
    B, S = x.shape
    H = pos_embedding.shape[1]
    rows_per_worker = S // _NW
    n_chunks = rows_per_worker // _CH
    mesh = plsc.VectorSubcoreMesh(core_axis_name="c", subcore_axis_name="s")
    body = functools.partial(_sc_body, B, n_chunks, rows_per_worker)
    k = functools.partial(
        pl.kernel,
        mesh=mesh,
        out_type=jax.ShapeDtypeStruct((B, S, H), pos_embedding.dtype),
        scratch_types=[
            pltpu.VMEM((_NSLOTS, _CH, H), pos_embedding.dtype),
            pltpu.SemaphoreType.DMA,
            pltpu.SemaphoreType.DMA,
        ],
    )(body)
    return k(pos_embedding)


# speedup vs baseline: 1.0452x; 1.0032x over previous
import functools
import jax
import jax.numpy as jnp
from jax import lax
from jax.experimental import pallas as pl
from jax.experimental.pallas import tpu as pltpu
from jax.experimental.pallas import tpu_sc as plsc

_NW = 32
_CH = 32
_NSLOTS = 3
_PF = 3


def _sc_body(batch, n_chunks, rows_per_worker, table_hbm, out_hbm, buf,
             in_sem, out_sem):
    wid = lax.axis_index("s") * 2 + lax.axis_index("c")
    base = wid * rows_per_worker

    def in_copy(c):
        return pltpu.make_async_copy(
            table_hbm.at[pl.ds(base + c * _CH, _CH), :],
            buf.at[c % _NSLOTS], in_sem,
        )

    def out_copy(c, b):
        return pltpu.make_async_copy(
            buf.at[c % _NSLOTS],
            out_hbm.at[b, pl.ds(base + c * _CH, _CH), :], out_sem,
        )

    for c in range(min(_PF, n_chunks)):
        in_copy(c).start()
    for c in range(n_chunks):
        in_copy(c).wait()
        for b in range(batch):
            out_copy(c, b).start()
        nxt = c + _PF
        if nxt < n_chunks:
            old = nxt - _NSLOTS
            if old >= 0:
                for b in range(batch):
                    out_copy(old, b).wait()
            in_copy(nxt).start()
    for c in range(max(0, n_chunks - _NSLOTS), n_chunks):
        for b in range(batch):
            out_copy(c, b).wait()


def kernel(x, pos_embedding):
    B, S = x.shape
    H = pos_embedding.shape[1]
    rows_per_worker = S // _NW
    n_chunks = rows_per_worker // _CH
    mesh = plsc.VectorSubcoreMesh(core_axis_name="c", subcore_axis_name="s")
    body = functools.partial(_sc_body, B, n_chunks, rows_per_worker)
    k = functools.partial(
        pl.kernel,
        mesh=mesh,
        out_type=jax.ShapeDtypeStruct((B, S, H), pos_embedding.dtype),
        scratch_types=[
            pltpu.VMEM((_NSLOTS, _CH, H), pos_embedding.dtype),
            pltpu.SemaphoreType.DMA,
            pltpu.SemaphoreType.DMA,
        ],
    )(body)
    return k(pos_embedding)
